# Initial kernel scaffold; baseline (speedup 1.0000x reference)
import jax
import jax.numpy as jnp
from jax.experimental import pallas as pl


def kernel(x, feature1, pos_emb, edge_index, batch, params):
    raise NotImplementedError("write your pallas kernel here")


# speedup vs baseline: 2.0574x; 2.0574x over previous
import functools

import jax
import jax.numpy as jnp
from jax import lax
from jax.experimental import pallas as pl
from jax.experimental.pallas import tpu as pltpu
from jax.experimental.pallas import tpu_sc as plsc

N = 10000
E = 320000
H = 128
MID = 64
F1 = 12
PE = 16

NC = 2
NS = 16
C = 80
CPT = E // C // NS
EPT = E // NS
NPAD = 10240
RPT = NPAD // NS
ZR = 128

f32 = jnp.float32


def _mm(a, w):
    return lax.dot_general(a, w, (((1,), (1,)), ((), ())),
                           preferred_element_type=f32)


def _swish(y):
    return y * jax.nn.sigmoid(y)


def _xlin_body(x_ref, w1, b1, w2, b2, o1, o2):
    x = x_ref[...]
    y1 = _mm(x, w1[...]) + b1[...]
    o1[...] = _swish(y1)
    y2 = _mm(x, w2[...]) + b2[...]
    o2[...] = _swish(y2)


def _xlin_call(x, w1, b1, w2, b2):
    BN = 2000
    full = pl.BlockSpec((H, H), lambda i: (0, 0))
    bias = pl.BlockSpec((1, H), lambda i: (0, 0))
    return pl.pallas_call(
        _xlin_body,
        grid=(N // BN,),
        in_specs=[pl.BlockSpec((BN, H), lambda i: (i, 0)), full, bias, full, bias],
        out_specs=[pl.BlockSpec((BN, H), lambda i: (i, 0))] * 2,
        out_shape=[jax.ShapeDtypeStruct((N, H), f32)] * 2,
    )(x, w1, b1, w2, b2)


def _feat_body(fe_ref, pe_ref, wa1, wb1, wa2, wb2, o1, o2):
    m1 = _mm(wb1[...], wa1[...].T)
    o1[...] = _mm(fe_ref[...], m1)
    m2 = _mm(wb2[...], wa2[...].T)
    o2[...] = _mm(pe_ref[...], m2)


def _feat_call(feature1, pos_emb, wa1, wb1, wa2, wb2):
    BE = 4000
    return pl.pallas_call(
        _feat_body,
        grid=(E // BE,),
        in_specs=[
            pl.BlockSpec((BE, F1), lambda i: (i, 0)),
            pl.BlockSpec((BE, PE), lambda i: (i, 0)),
            pl.BlockSpec((MID, F1), lambda i: (0, 0)),
            pl.BlockSpec((H, MID), lambda i: (0, 0)),
            pl.BlockSpec((MID, PE), lambda i: (0, 0)),
            pl.BlockSpec((H, MID), lambda i: (0, 0)),
        ],
        out_specs=[pl.BlockSpec((BE, H), lambda i: (i, 0))] * 2,
        out_shape=[jax.ShapeDtypeStruct((E, H), f32)] * 2,
    )(feature1, pos_emb, wa1, wb1, wa2, wb2)


def _sc_body(xw, f1, f2, src, dst, agg1, agg2,
             shared, idx_s, idx_d, rows, fbuf, sem):
    c = lax.axis_index("c")
    s = lax.axis_index("s")

    def zrow(i, _):
        for j in range(H // 16):
            rows[i, pl.ds(j * 16, 16)] = jnp.zeros((16,), f32)
        return 0
    lax.fori_loop(0, C, zrow, 0)
    for k in range(RPT // C):
        pltpu.sync_copy(rows, shared.at[pl.ds(s * RPT + k * C, C)])
    plsc.subcore_barrier()

    def edge_loop(f_hbm):
        def chunk(i, _):
            ebase = s * EPT + i * C
            pltpu.sync_copy(src.at[pl.ds(ebase, C)], idx_s)
            pltpu.sync_copy(dst.at[pl.ds(ebase, C)], idx_d)
            gat = pltpu.async_copy(xw.at[idx_s], rows, sem)
            pltpu.sync_copy(f_hbm.at[pl.ds(ebase, C)], fbuf)
            gat.wait()

            def mulrow(e, _):
                for j in range(H // 16):
                    sl = pl.ds(j * 16, 16)
                    rows[e, sl] = rows[e, sl] * fbuf[e, sl]
                return 0
            lax.fori_loop(0, C, mulrow, 0)
            pltpu.sync_copy(rows, shared.at[idx_d], add=True)
            return 0
        lax.fori_loop(0, CPT, chunk, 0)

    @pl.when(c == 0)
    def _():
        edge_loop(f1)

    @pl.when(c == 1)
    def _():
        edge_loop(f2)

    plsc.subcore_barrier()

    @pl.when(c == 0)
    def _():
        pltpu.sync_copy(shared.at[pl.ds(s * RPT, RPT)],
                        agg1.at[pl.ds(s * RPT, RPT)])

    @pl.when(c == 1)
    def _():
        pltpu.sync_copy(shared.at[pl.ds(s * RPT, RPT)],
                        agg2.at[pl.ds(s * RPT, RPT)])


def _sc_call(xw, f1, f2, src2, dst2):
    mesh = plsc.VectorSubcoreMesh(core_axis_name="c", subcore_axis_name="s")
    return pl.kernel(
        _sc_body,
        out_type=[jax.ShapeDtypeStruct((NPAD, H), f32)] * 2,
        mesh=mesh,
        scratch_types=[
            pltpu.VMEM_SHARED((NPAD, H), f32),
            pltpu.VMEM((C,), jnp.int32),
            pltpu.VMEM((C,), jnp.int32),
            pltpu.VMEM((C, H), f32),
            pltpu.VMEM((C, H), f32),
            pltpu.SemaphoreType.DMA,
        ],
    )(xw, f1, f2, src2, dst2)


def _tail_body(a1, a2, x1, x2,
               wc1l, bc1l, wc1r, wl1, bl1,
               wc2l, bc2l, wc2r, wl2, bl2,
               wc0a, wc0b, bc0, wca1, bca1, wca2, bca2,
               wl0, bl0, wll1, bll1, wf, bf, out):
    xl1 = x1[...]
    h1 = _mm(a1[...], wc1l[...]) + bc1l[...] + _mm(xl1, wc1r[...])
    h1 = _swish(_mm(h1, wl1[...]) + bl1[...])
    h2 = _mm(a2[...], wc2l[...]) + bc2l[...] + _mm(xl1, wc2r[...])
    h2 = _swish(_mm(h2, wl2[...]) + bl2[...])
    h = _swish(_mm(h1, wc0a[...]) + _mm(h2, wc0b[...]) + bc0[...])
    h = _swish(_mm(h, wca1[...]) + bca1[...])
    h = _swish(_mm(h, wca2[...]) + bca2[...])
    h = h + x2[...]
    h = _swish(_mm(h, wl0[...]) + bl0[...])
    h = _swish(_mm(h, wll1[...]) + bll1[...])
    out[...] = _mm(h, wf[...]) + bf[...]


def _tail_call(a1, a2, x1, x2, weights):
    BN = 2000
    blk = pl.BlockSpec((BN, H), lambda i: (i, 0))
    full = pl.BlockSpec((H, H), lambda i: (0, 0))
    bias = pl.BlockSpec((1, H), lambda i: (0, 0))
    wspecs = []
    for w in weights:
        wspecs.append(bias if w.shape[0] == 1 else full)
    return pl.pallas_call(
        _tail_body,
        grid=(N // BN,),
        in_specs=[blk, blk, blk, blk] + wspecs,
        out_specs=blk,
        out_shape=jax.ShapeDtypeStruct((N, H), f32),
    )(a1, a2, x1, x2, *weights)


def kernel(x, feature1, pos_emb, edge_index, batch, params):
    p = params
    src2 = edge_index[0].astype(jnp.int32)
    dst2 = edge_index[1].astype(jnp.int32)

    def b(name):
        return p[name].reshape(1, H)

    xl1, xl2 = _xlin_call(x, p['W_lin_1'], b('b_lin_1'),
                          p['W_lin_2'], b('b_lin_2'))
    f1, f2 = _feat_call(feature1, pos_emb,
                        p['Wf1_a'], p['Wf1_b'], p['Wf2_a'], p['Wf2_b'])
    agg1, agg2 = _sc_call(xl1, f1, f2, src2, dst2)

    weights = [
        p['Wc1_l'], b('bc1_l'), p['Wc1_r'], p['W_lin1'], b('b_lin1'),
        p['Wc2_l'], b('bc2_l'), p['Wc2_r'], p['W_lin2'], b('b_lin2'),
        p['W_cat0'][:, :H], p['W_cat0'][:, H:], b('b_cat0'),
        p['W_cat1'], b('b_cat1'), p['W_cat2'], b('b_cat2'),
        p['W_l0'], b('b_l0'), p['W_l1'], b('b_l1'),
        p['W_final'], b('b_final'),
    ]
    return _tail_call(agg1, agg2, xl1, xl2, weights)


# speedup vs baseline: 3.1391x; 1.5257x over previous
import functools

import jax
import jax.numpy as jnp
from jax import lax
from jax.experimental import pallas as pl
from jax.experimental.pallas import tpu as pltpu
from jax.experimental.pallas import tpu_sc as plsc

N = 10000
E = 320000
H = 128
MID = 64
F1 = 12
PE = 16

NC = 2
NS = 16
C = 80
CPT = E // C // NS
K = 10
BPT = CPT // K
EPT = E // NS
NPAD = 10240
RPT = NPAD // NS
ZR = 128

f32 = jnp.float32


def _mm(a, w):
    return lax.dot_general(a, w, (((1,), (1,)), ((), ())),
                           preferred_element_type=f32)


def _swish(y):
    return y * jax.nn.sigmoid(y)


def _xlin_body(x_ref, w1, b1, w2, b2, o1, o2):
    x = x_ref[...]
    y1 = _mm(x, w1[...]) + b1[...]
    o1[...] = _swish(y1)
    y2 = _mm(x, w2[...]) + b2[...]
    o2[...] = _swish(y2)


def _xlin_call(x, w1, b1, w2, b2):
    BN = 2000
    full = pl.BlockSpec((H, H), lambda i: (0, 0))
    bias = pl.BlockSpec((1, H), lambda i: (0, 0))
    return pl.pallas_call(
        _xlin_body,
        grid=(N // BN,),
        in_specs=[pl.BlockSpec((BN, H), lambda i: (i, 0)), full, bias, full, bias],
        out_specs=[pl.BlockSpec((BN, H), lambda i: (i, 0))] * 2,
        out_shape=[jax.ShapeDtypeStruct((N, H), f32)] * 2,
    )(x, w1, b1, w2, b2)


def _feat_body(fe_ref, pe_ref, wa1, wb1, wa2, wb2, o1, o2):
    m1 = _mm(wb1[...], wa1[...].T)
    o1[...] = _mm(fe_ref[...], m1)
    m2 = _mm(wb2[...], wa2[...].T)
    o2[...] = _mm(pe_ref[...], m2)


def _feat_call(feature1, pos_emb, wa1, wb1, wa2, wb2):
    BE = 4000
    return pl.pallas_call(
        _feat_body,
        grid=(E // BE,),
        in_specs=[
            pl.BlockSpec((BE, F1), lambda i: (i, 0)),
            pl.BlockSpec((BE, PE), lambda i: (i, 0)),
            pl.BlockSpec((MID, F1), lambda i: (0, 0)),
            pl.BlockSpec((H, MID), lambda i: (0, 0)),
            pl.BlockSpec((MID, PE), lambda i: (0, 0)),
            pl.BlockSpec((H, MID), lambda i: (0, 0)),
        ],
        out_specs=[pl.BlockSpec((BE, H), lambda i: (i, 0))] * 2,
        out_shape=[jax.ShapeDtypeStruct((E, H), f32)] * 2,
    )(feature1, pos_emb, wa1, wb1, wa2, wb2)


def _sc_body(xw, f1, f2, src3, dst3, agg1, agg2,
             shared, idx_s, idx_d, rows0, rows1, fb0, fb1,
             gs0, gs1, fs0, fs1, ss0, ss1):
    c = lax.axis_index("c")
    s = lax.axis_index("s")
    rows = [rows0, rows1]
    fbuf = [fb0, fb1]
    gsem = [gs0, gs1]
    fsem = [fs0, fs1]
    ssem = [ss0, ss1]

    def zrow(i, _):
        for j in range(H // 16):
            rows0[i, pl.ds(j * 16, 16)] = jnp.zeros((16,), f32)
        return 0
    lax.fori_loop(0, C, zrow, 0)
    for k in range(RPT // C):
        pltpu.sync_copy(rows0, shared.at[pl.ds(s * RPT + k * C, C)])
    plsc.subcore_barrier()

    def edge_loop(f_hbm):
        def block(i, _):
            blk = s * BPT + i
            ebase0 = blk * K * C
            pltpu.sync_copy(src3.at[blk], idx_s)
            pltpu.sync_copy(dst3.at[blk], idx_d)

            def gat(j):
                return pltpu.async_copy(xw.at[idx_s.at[j]], rows[j % 2],
                                        gsem[j % 2])

            def fld(j):
                return pltpu.async_copy(
                    f_hbm.at[pl.ds(ebase0 + j * C, C)], fbuf[j % 2],
                    fsem[j % 2])

            g = [None] * K
            fd = [None] * K
            sd = [None] * K
            g[0] = gat(0)
            fd[0] = fld(0)
            g[1] = gat(1)
            for j in range(K):
                b = j % 2
                g[j].wait()
                fd[j].wait()
                if j + 1 < K:
                    if j >= 1:
                        sd[j - 1].wait()
                    fd[j + 1] = fld(j + 1)

                fb = fbuf[b]
                rw = rows[b]

                @plsc.parallel_loop(0, C, step=1, unroll=2)
                def _(e):
                    for h8 in range(H // 16):
                        sl = pl.ds(h8 * 16, 16)
                        fb[e, sl] = fb[e, sl] * rw[e, sl]

                sd[j] = pltpu.async_copy(fb, shared.at[idx_d.at[j]],
                                         ssem[b], add=True)
                if j + 2 < K:
                    g[j + 2] = gat(j + 2)
            sd[K - 2].wait()
            sd[K - 1].wait()
            return 0
        lax.fori_loop(0, BPT, block, 0)

    @pl.when(c == 0)
    def _():
        edge_loop(f1)

    @pl.when(c == 1)
    def _():
        edge_loop(f2)

    plsc.subcore_barrier()

    @pl.when(c == 0)
    def _():
        pltpu.sync_copy(shared.at[pl.ds(s * RPT, RPT)],
                        agg1.at[pl.ds(s * RPT, RPT)])

    @pl.when(c == 1)
    def _():
        pltpu.sync_copy(shared.at[pl.ds(s * RPT, RPT)],
                        agg2.at[pl.ds(s * RPT, RPT)])


def _sc_call(xw, f1, f2, src2, dst2):
    mesh = plsc.VectorSubcoreMesh(core_axis_name="c", subcore_axis_name="s")
    return pl.kernel(
        _sc_body,
        out_type=[jax.ShapeDtypeStruct((NPAD, H), f32)] * 2,
        mesh=mesh,
        scratch_types=[
            pltpu.VMEM_SHARED((NPAD, H), f32),
            pltpu.VMEM((K, C), jnp.int32),
            pltpu.VMEM((K, C), jnp.int32),
            pltpu.VMEM((C, H), f32),
            pltpu.VMEM((C, H), f32),
            pltpu.VMEM((C, H), f32),
            pltpu.VMEM((C, H), f32),
        ] + [pltpu.SemaphoreType.DMA] * 6,
    )(xw, f1, f2, src2, dst2)


def _tail_body(a1, a2, x1, x2,
               wc1l, bc1l, wc1r, wl1, bl1,
               wc2l, bc2l, wc2r, wl2, bl2,
               wc0a, wc0b, bc0, wca1, bca1, wca2, bca2,
               wl0, bl0, wll1, bll1, wf, bf, out):
    xl1 = x1[...]
    h1 = _mm(a1[...], wc1l[...]) + bc1l[...] + _mm(xl1, wc1r[...])
    h1 = _swish(_mm(h1, wl1[...]) + bl1[...])
    h2 = _mm(a2[...], wc2l[...]) + bc2l[...] + _mm(xl1, wc2r[...])
    h2 = _swish(_mm(h2, wl2[...]) + bl2[...])
    h = _swish(_mm(h1, wc0a[...]) + _mm(h2, wc0b[...]) + bc0[...])
    h = _swish(_mm(h, wca1[...]) + bca1[...])
    h = _swish(_mm(h, wca2[...]) + bca2[...])
    h = h + x2[...]
    h = _swish(_mm(h, wl0[...]) + bl0[...])
    h = _swish(_mm(h, wll1[...]) + bll1[...])
    out[...] = _mm(h, wf[...]) + bf[...]


def _tail_call(a1, a2, x1, x2, weights):
    BN = 2000
    blk = pl.BlockSpec((BN, H), lambda i: (i, 0))
    full = pl.BlockSpec((H, H), lambda i: (0, 0))
    bias = pl.BlockSpec((1, H), lambda i: (0, 0))
    wspecs = []
    for w in weights:
        wspecs.append(bias if w.shape[0] == 1 else full)
    return pl.pallas_call(
        _tail_body,
        grid=(N // BN,),
        in_specs=[blk, blk, blk, blk] + wspecs,
        out_specs=blk,
        out_shape=jax.ShapeDtypeStruct((N, H), f32),
    )(a1, a2, x1, x2, *weights)


def kernel(x, feature1, pos_emb, edge_index, batch, params):
    p = params
    src2 = edge_index[0].astype(jnp.int32).reshape(NS * BPT, K, C)
    dst2 = edge_index[1].astype(jnp.int32).reshape(NS * BPT, K, C)

    def b(name):
        return p[name].reshape(1, H)

    xl1, xl2 = _xlin_call(x, p['W_lin_1'], b('b_lin_1'),
                          p['W_lin_2'], b('b_lin_2'))
    f1, f2 = _feat_call(feature1, pos_emb,
                        p['Wf1_a'], p['Wf1_b'], p['Wf2_a'], p['Wf2_b'])
    agg1, agg2 = _sc_call(xl1, f1, f2, src2, dst2)

    weights = [
        p['Wc1_l'], b('bc1_l'), p['Wc1_r'], p['W_lin1'], b('b_lin1'),
        p['Wc2_l'], b('bc2_l'), p['Wc2_r'], p['W_lin2'], b('b_lin2'),
        p['W_cat0'][:, :H], p['W_cat0'][:, H:], b('b_cat0'),
        p['W_cat1'], b('b_cat1'), p['W_cat2'], b('b_cat2'),
        p['W_l0'], b('b_l0'), p['W_l1'], b('b_l1'),
        p['W_final'], b('b_final'),
    ]
    return _tail_call(agg1, agg2, xl1, xl2, weights)
